# Initial kernel scaffold; baseline (speedup 1.0000x reference)
import jax
import jax.numpy as jnp
from jax.experimental import pallas as pl


def kernel(inputs, table):
    raise NotImplementedError("write your pallas kernel here")


# speedup vs baseline: 9.1798x; 9.1798x over previous
import functools

import jax
import jax.numpy as jnp
from jax import lax
from jax.experimental import pallas as pl
from jax.experimental.pallas import tpu as pltpu
from jax.experimental.pallas import tpu_sc as plsc

NC = 2
NS = 16
NW = NC * NS
CHUNK = 128
NBUF = 4


@functools.lru_cache(maxsize=None)
def _build_gather(n_chunks_total, chunk, d):
  n_chunks_w = n_chunks_total // NW
  mesh = plsc.VectorSubcoreMesh(
      core_axis_name="c", subcore_axis_name="s",
      num_cores=NC, num_subcores=NS)

  def body(idx_hbm, table_hbm, out_hbm, idx_v, rows_v, *sems):
    wid = lax.axis_index("s") * NC + lax.axis_index("c")
    first = wid * n_chunks_w
    pltpu.sync_copy(idx_hbm.at[pl.ds(first, n_chunks_w)], idx_v)

    def fire(j, b):
      pltpu.async_copy(table_hbm.at[idx_v.at[j]], rows_v.at[b], sems[b])

    def wait(b):
      pltpu.make_async_copy(table_hbm.at[idx_v.at[0]], rows_v.at[b],
                            sems[b]).wait()

    for b in range(NBUF):
      fire(b, b)

    def group(g, _):
      for b in range(NBUF):
        j = g * NBUF + b
        wait(b)
        pltpu.sync_copy(rows_v.at[b],
                        out_hbm.at[pl.ds((first + j) * chunk, chunk)])
        jn = j + NBUF

        @pl.when(jn < n_chunks_w)
        def _():
          fire(jn, b)

      return 0

    lax.fori_loop(0, n_chunks_w // NBUF, group, 0)

  return pl.kernel(
      body,
      out_type=jax.ShapeDtypeStruct((n_chunks_total * chunk, d),
                                    jnp.float32),
      mesh=mesh,
      scratch_types=[
          pltpu.VMEM((n_chunks_w, chunk), jnp.int32),
          pltpu.VMEM((NBUF, chunk, d), jnp.float32),
      ] + [pltpu.SemaphoreType.DMA] * NBUF,
  )


def kernel(inputs, table):
  batch, hist = inputs.shape
  _, d = table.shape
  total = batch * hist
  grain = NW * CHUNK
  padded = (total + grain - 1) // grain * grain
  idx = inputs.reshape(total).astype(jnp.int32)
  if padded != total:
    idx = jnp.concatenate([idx, jnp.zeros(padded - total, jnp.int32)])
  idx = idx.reshape(padded // CHUNK, CHUNK)
  out = _build_gather(padded // CHUNK, CHUNK, d)(idx, table)
  return out[:total].reshape(batch, hist, d)
